# Initial kernel scaffold; baseline (speedup 1.0000x reference)
import jax
import jax.numpy as jnp
from jax.experimental import pallas as pl


def kernel(x, edge_index, W0, W1, W2, W3, W4, b0, b1, b2, b3, b4):
    raise NotImplementedError("write your pallas kernel here")


# speedup vs baseline: 5.5473x; 5.5473x over previous
import functools

import jax
import jax.numpy as jnp
from jax import lax
from jax.experimental import pallas as pl
from jax.experimental.pallas import tpu as pltpu
from jax.experimental.pallas import tpu_sc as plsc

N = 10000
E = 320000
D = 128
DH = D // 2
NUM_LAYERS = 5

NC = 2
NS = 16
NW = NC * NS
CHUNK = 128
EPAD = 327680
EB = EPAD // CHUNK
RB = EB // NW
NT = N + 16
NA = 10240
ROWS_PER_TILE = NA // NS
DW = 16

_mesh = plsc.VectorSubcoreMesh(core_axis_name="c", subcore_axis_name="s")


@functools.partial(
    pl.kernel,
    out_type=jax.ShapeDtypeStruct((NC, NA, DW), jnp.float32),
    mesh=_mesh,
    scratch_types=[
        pltpu.VMEM((RB, CHUNK), jnp.int32),
        pltpu.VMEM((CHUNK, DW), jnp.float32),
        pltpu.VMEM_SHARED((NA, DW), jnp.float32),
    ],
    compiler_params=pltpu.CompilerParams(use_tc_tiling_on_sc=False),
)
def _degree_call(dst_hbm, ones_hbm, zeros_hbm, out_hbm, dstv, onesv, acc):
    cid = lax.axis_index("c")
    sid = lax.axis_index("s")
    wid = cid * NS + sid

    pltpu.sync_copy(ones_hbm, onesv)

    @pl.loop(0, ROWS_PER_TILE // CHUNK)
    def _zero(i):
        pltpu.sync_copy(
            zeros_hbm, acc.at[pl.ds(sid * ROWS_PER_TILE + i * CHUNK, CHUNK)]
        )

    pltpu.sync_copy(dst_hbm.at[pl.ds(wid * RB, RB)], dstv)
    plsc.subcore_barrier()

    @pl.loop(0, RB)
    def _scat(g):
        pltpu.sync_copy(onesv, acc.at[dstv.at[g]], add=True)

    plsc.subcore_barrier()
    pltpu.sync_copy(
        acc.at[pl.ds(sid * ROWS_PER_TILE, ROWS_PER_TILE)],
        out_hbm.at[cid, pl.ds(sid * ROWS_PER_TILE, ROWS_PER_TILE)],
    )


@functools.partial(
    pl.kernel,
    out_type=(
        jax.ShapeDtypeStruct((NC, NA, DH), jnp.float32),
        jax.ShapeDtypeStruct((NC, NA, DH), jnp.float32),
    ),
    mesh=_mesh,
    scratch_types=[
        pltpu.VMEM((RB, CHUNK), jnp.int32),
        pltpu.VMEM((RB, CHUNK), jnp.int32),
        pltpu.VMEM((CHUNK, DH), jnp.float32),
        pltpu.VMEM((CHUNK, DH), jnp.float32),
        pltpu.VMEM_SHARED((NA, DH), jnp.float32),
        pltpu.SemaphoreType.DMA,
        pltpu.SemaphoreType.DMA,
    ],
    compiler_params=pltpu.CompilerParams(use_tc_tiling_on_sc=False),
)
def _agg_call(hp0_hbm, hp1_hbm, src_hbm, dst_hbm, zeros_hbm,
              out0_hbm, out1_hbm,
              srcv, dstv, rb0, rb1, acc, sem0, sem1):
    cid = lax.axis_index("c")
    sid = lax.axis_index("s")
    wid = cid * NS + sid

    pltpu.sync_copy(src_hbm.at[pl.ds(wid * RB, RB)], srcv)
    pltpu.sync_copy(dst_hbm.at[pl.ds(wid * RB, RB)], dstv)

    for hp_hbm, out_hbm in ((hp0_hbm, out0_hbm), (hp1_hbm, out1_hbm)):
        @pl.loop(0, ROWS_PER_TILE // CHUNK)
        def _zero(i):
            pltpu.sync_copy(
                zeros_hbm, acc.at[pl.ds(sid * ROWS_PER_TILE + i * CHUNK, CHUNK)]
            )

        plsc.subcore_barrier()

        pltpu.async_copy(hp_hbm.at[srcv.at[0]], rb0, sem0)

        @pl.loop(0, RB, step=2)
        def _edges(g):
            pltpu.make_async_copy(hp_hbm.at[srcv.at[g]], rb0, sem0).wait()
            pltpu.async_copy(hp_hbm.at[srcv.at[g + 1]], rb1, sem1)
            pltpu.sync_copy(rb0, acc.at[dstv.at[g]], add=True)
            pltpu.make_async_copy(hp_hbm.at[srcv.at[g + 1]], rb1, sem1).wait()

            @pl.when(g + 2 < RB)
            def _next():
                pltpu.async_copy(hp_hbm.at[srcv.at[g + 2]], rb0, sem0)

            pltpu.sync_copy(rb1, acc.at[dstv.at[g + 1]], add=True)

        plsc.subcore_barrier()
        pltpu.sync_copy(
            acc.at[pl.ds(sid * ROWS_PER_TILE, ROWS_PER_TILE)],
            out_hbm.at[cid, pl.ds(sid * ROWS_PER_TILE, ROWS_PER_TILE)],
        )


ROW_BLK = 2000
GRID = N // ROW_BLK


def _first_body(deg_ref, x_ref, w_ref, dis_ref, hp_ref):
    d = deg_ref[0] + deg_ref[1] + 1.0
    dis = lax.rsqrt(d)
    dis_ref[...] = dis[:, 0:8]
    h = jnp.dot(x_ref[...], w_ref[...], preferred_element_type=jnp.float32,
                precision=lax.Precision.HIGHEST)
    hp_ref[...] = h * dis[:, 0:1]


def _tc_first(deg2, x, w0):
    return pl.pallas_call(
        _first_body,
        grid=(GRID,),
        in_specs=[
            pl.BlockSpec((NC, ROW_BLK, DW), lambda r: (0, r, 0)),
            pl.BlockSpec((ROW_BLK, D), lambda r: (r, 0)),
            pl.BlockSpec((D, D), lambda r: (0, 0)),
        ],
        out_specs=[
            pl.BlockSpec((ROW_BLK, 8), lambda r: (r, 0)),
            pl.BlockSpec((ROW_BLK, D), lambda r: (r, 0)),
        ],
        out_shape=[
            jax.ShapeDtypeStruct((N, 8), jnp.float32),
            jax.ShapeDtypeStruct((N, D), jnp.float32),
        ],
    )(deg2, x, w0)


def _log_softmax(a):
    m = jnp.max(a, axis=1, keepdims=True)
    return a - m - jnp.log(jnp.sum(jnp.exp(a - m), axis=1, keepdims=True))


def _combine(p0_ref, p1_ref, hp_ref, dis_ref, b_ref):
    s = jnp.concatenate([p0_ref[0] + p0_ref[1], p1_ref[0] + p1_ref[1]], axis=1)
    return (s + hp_ref[...]) * dis_ref[:, 0:1] + b_ref[...]


def _mid_body(p0_ref, p1_ref, hp_ref, dis_ref, b_ref, w_ref, out_ref, hpn_ref):
    t = _combine(p0_ref, p1_ref, hp_ref, dis_ref, b_ref)
    a = jnp.maximum(t, 0.0)
    out_ref[...] = _log_softmax(a)
    hpn_ref[...] = (
        jnp.dot(a, w_ref[...], preferred_element_type=jnp.float32,
                precision=lax.Precision.HIGHEST)
        * dis_ref[:, 0:1]
    )


def _tc_mid(p0, p1, hp, dis8, b, w_next):
    return pl.pallas_call(
        _mid_body,
        grid=(GRID,),
        in_specs=[
            pl.BlockSpec((NC, ROW_BLK, DH), lambda r: (0, r, 0)),
            pl.BlockSpec((NC, ROW_BLK, DH), lambda r: (0, r, 0)),
            pl.BlockSpec((ROW_BLK, D), lambda r: (r, 0)),
            pl.BlockSpec((ROW_BLK, 8), lambda r: (r, 0)),
            pl.BlockSpec((1, D), lambda r: (0, 0)),
            pl.BlockSpec((D, D), lambda r: (0, 0)),
        ],
        out_specs=[
            pl.BlockSpec((ROW_BLK, D), lambda r: (r, 0)),
            pl.BlockSpec((ROW_BLK, D), lambda r: (r, 0)),
        ],
        out_shape=[
            jax.ShapeDtypeStruct((N, D), jnp.float32),
            jax.ShapeDtypeStruct((N, D), jnp.float32),
        ],
    )(p0, p1, hp, dis8, b, w_next)


def _last_body(p0_ref, p1_ref, hp_ref, dis_ref, b_ref, out_ref):
    t = _combine(p0_ref, p1_ref, hp_ref, dis_ref, b_ref)
    out_ref[...] = _log_softmax(t)


def _tc_last(p0, p1, hp, dis8, b):
    return pl.pallas_call(
        _last_body,
        grid=(GRID,),
        in_specs=[
            pl.BlockSpec((NC, ROW_BLK, DH), lambda r: (0, r, 0)),
            pl.BlockSpec((NC, ROW_BLK, DH), lambda r: (0, r, 0)),
            pl.BlockSpec((ROW_BLK, D), lambda r: (r, 0)),
            pl.BlockSpec((ROW_BLK, 8), lambda r: (r, 0)),
            pl.BlockSpec((1, D), lambda r: (0, 0)),
        ],
        out_specs=pl.BlockSpec((ROW_BLK, D), lambda r: (r, 0)),
        out_shape=jax.ShapeDtypeStruct((N, D), jnp.float32),
    )(p0, p1, hp, dis8, b)


def kernel(x, edge_index, W0, W1, W2, W3, W4, b0, b1, b2, b3, b4):
    Ws = [W0, W1, W2, W3, W4]
    bs = [b0, b1, b2, b3, b4]

    src = edge_index[0]
    dst = edge_index[1]
    pad = jnp.full((EPAD - E,), N, dtype=jnp.int32)
    src2 = jnp.concatenate([src, pad]).reshape(EB, CHUNK)
    dst2 = jnp.concatenate([dst, pad]).reshape(EB, CHUNK)

    ones_blk = jnp.ones((CHUNK, DW), dtype=jnp.float32)
    zeros_blk = jnp.zeros((CHUNK, DW), dtype=jnp.float32)
    zeros_blk_h = jnp.zeros((CHUNK, DH), dtype=jnp.float32)
    deg2 = _degree_call(dst2, ones_blk, zeros_blk)
    dis8, hp = _tc_first(deg2, x, Ws[0])

    zrows = jnp.zeros((NT - N, D), dtype=jnp.float32)
    outs = []
    for i in range(NUM_LAYERS):
        hp_pad = jnp.concatenate([hp, zrows])
        p0, p1 = _agg_call(hp_pad[:, :DH], hp_pad[:, DH:], src2, dst2,
                           zeros_blk_h)
        b = bs[i].reshape(1, D)
        if i < NUM_LAYERS - 1:
            o, hp = _tc_mid(p0, p1, hp, dis8, b, Ws[i + 1])
        else:
            o = _tc_last(p0, p1, hp, dis8, b)
        outs.append(o)
    return tuple(outs)


# speedup vs baseline: 6.0689x; 1.0940x over previous
import functools

import jax
import jax.numpy as jnp
from jax import lax
from jax.experimental import pallas as pl
from jax.experimental.pallas import tpu as pltpu
from jax.experimental.pallas import tpu_sc as plsc

N = 10000
E = 320000
D = 128
DH = D // 2
NUM_LAYERS = 5

NC = 2
NS = 16
NW = NC * NS
CHUNK = 128
EPAD = 327680
EB = EPAD // CHUNK
RB = EB // NW
NT = N + 16
NA = 10240
ROWS_PER_TILE = NA // NS
DW = 16
DEPTH = 8
LOOK = 4

_mesh = plsc.VectorSubcoreMesh(core_axis_name="c", subcore_axis_name="s")


@functools.partial(
    pl.kernel,
    out_type=jax.ShapeDtypeStruct((NC, NA, DW), jnp.float32),
    mesh=_mesh,
    scratch_types=[
        pltpu.VMEM((RB, CHUNK), jnp.int32),
        pltpu.VMEM((CHUNK, DW), jnp.float32),
        pltpu.VMEM_SHARED((NA, DW), jnp.float32),
    ],
    compiler_params=pltpu.CompilerParams(use_tc_tiling_on_sc=False),
)
def _degree_call(dst_hbm, ones_hbm, zeros_hbm, out_hbm, dstv, onesv, acc):
    cid = lax.axis_index("c")
    sid = lax.axis_index("s")
    wid = cid * NS + sid

    pltpu.sync_copy(ones_hbm, onesv)

    @pl.loop(0, ROWS_PER_TILE // CHUNK)
    def _zero(i):
        pltpu.sync_copy(
            zeros_hbm, acc.at[pl.ds(sid * ROWS_PER_TILE + i * CHUNK, CHUNK)]
        )

    pltpu.sync_copy(dst_hbm.at[pl.ds(wid * RB, RB)], dstv)
    plsc.subcore_barrier()

    @pl.loop(0, RB)
    def _scat(g):
        pltpu.sync_copy(onesv, acc.at[dstv.at[g]], add=True)

    plsc.subcore_barrier()
    pltpu.sync_copy(
        acc.at[pl.ds(sid * ROWS_PER_TILE, ROWS_PER_TILE)],
        out_hbm.at[cid, pl.ds(sid * ROWS_PER_TILE, ROWS_PER_TILE)],
    )


@functools.partial(
    pl.kernel,
    out_type=(
        jax.ShapeDtypeStruct((NC, NA, DH), jnp.float32),
        jax.ShapeDtypeStruct((NC, NA, DH), jnp.float32),
    ),
    mesh=_mesh,
    scratch_types=[
        pltpu.VMEM((RB, CHUNK), jnp.int32),
        pltpu.VMEM((RB, CHUNK), jnp.int32),
        pltpu.VMEM((DEPTH, CHUNK, DH), jnp.float32),
        pltpu.VMEM_SHARED((NA, DH), jnp.float32),
        pltpu.SemaphoreType.DMA((DEPTH,)),
        pltpu.SemaphoreType.DMA((DEPTH,)),
    ],
    compiler_params=pltpu.CompilerParams(use_tc_tiling_on_sc=False),
)
def _agg_call(hp0_hbm, hp1_hbm, src_hbm, dst_hbm, zeros_hbm,
              out0_hbm, out1_hbm,
              srcv, dstv, rbuf, acc, gsem, ssem):
    cid = lax.axis_index("c")
    sid = lax.axis_index("s")
    wid = cid * NS + sid

    pltpu.sync_copy(src_hbm.at[pl.ds(wid * RB, RB)], srcv)
    pltpu.sync_copy(dst_hbm.at[pl.ds(wid * RB, RB)], dstv)

    for hp_hbm, out_hbm in ((hp0_hbm, out0_hbm), (hp1_hbm, out1_hbm)):
        @pl.loop(0, ROWS_PER_TILE // CHUNK)
        def _zero(i):
            pltpu.sync_copy(
                zeros_hbm, acc.at[pl.ds(sid * ROWS_PER_TILE + i * CHUNK, CHUNK)]
            )

        plsc.subcore_barrier()

        for k in range(LOOK):
            pltpu.async_copy(hp_hbm.at[srcv.at[k]], rbuf.at[k], gsem.at[k])

        @pl.loop(0, RB)
        def _edges(j):
            k = lax.rem(j, DEPTH)
            kf = lax.rem(j + LOOK, DEPTH)

            @pl.when(j + LOOK < RB)
            def _prefetch():
                @pl.when(j + LOOK >= DEPTH)
                def _freebuf():
                    pltpu.make_async_copy(
                        rbuf.at[kf], acc.at[dstv.at[0]], ssem.at[kf]
                    ).wait()

                pltpu.async_copy(
                    hp_hbm.at[srcv.at[j + LOOK]], rbuf.at[kf], gsem.at[kf]
                )

            pltpu.make_async_copy(
                hp_hbm.at[srcv.at[j]], rbuf.at[k], gsem.at[k]
            ).wait()
            pltpu.async_copy(rbuf.at[k], acc.at[dstv.at[j]], ssem.at[k],
                             add=True)

        for k in range(DEPTH):
            pltpu.make_async_copy(
                rbuf.at[k], acc.at[dstv.at[0]], ssem.at[k]
            ).wait()

        plsc.subcore_barrier()
        pltpu.sync_copy(
            acc.at[pl.ds(sid * ROWS_PER_TILE, ROWS_PER_TILE)],
            out_hbm.at[cid, pl.ds(sid * ROWS_PER_TILE, ROWS_PER_TILE)],
        )


ROW_BLK = 2000
GRID = N // ROW_BLK


def _first_body(deg_ref, x_ref, w_ref, dis_ref, hp_ref):
    d = deg_ref[0] + deg_ref[1] + 1.0
    dis = lax.rsqrt(d)
    dis_ref[...] = dis[:, 0:8]
    h = jnp.dot(x_ref[...], w_ref[...], preferred_element_type=jnp.float32,
                precision=lax.Precision.HIGHEST)
    hp_ref[...] = h * dis[:, 0:1]


def _tc_first(deg2, x, w0):
    return pl.pallas_call(
        _first_body,
        grid=(GRID,),
        in_specs=[
            pl.BlockSpec((NC, ROW_BLK, DW), lambda r: (0, r, 0)),
            pl.BlockSpec((ROW_BLK, D), lambda r: (r, 0)),
            pl.BlockSpec((D, D), lambda r: (0, 0)),
        ],
        out_specs=[
            pl.BlockSpec((ROW_BLK, 8), lambda r: (r, 0)),
            pl.BlockSpec((ROW_BLK, D), lambda r: (r, 0)),
        ],
        out_shape=[
            jax.ShapeDtypeStruct((N, 8), jnp.float32),
            jax.ShapeDtypeStruct((N, D), jnp.float32),
        ],
    )(deg2, x, w0)


def _log_softmax(a):
    m = jnp.max(a, axis=1, keepdims=True)
    return a - m - jnp.log(jnp.sum(jnp.exp(a - m), axis=1, keepdims=True))


def _combine(p0_ref, p1_ref, hp_ref, dis_ref, b_ref):
    s = jnp.concatenate([p0_ref[0] + p0_ref[1], p1_ref[0] + p1_ref[1]], axis=1)
    return (s + hp_ref[...]) * dis_ref[:, 0:1] + b_ref[...]


def _mid_body(p0_ref, p1_ref, hp_ref, dis_ref, b_ref, w_ref, out_ref, hpn_ref):
    t = _combine(p0_ref, p1_ref, hp_ref, dis_ref, b_ref)
    a = jnp.maximum(t, 0.0)
    out_ref[...] = _log_softmax(a)
    hpn_ref[...] = (
        jnp.dot(a, w_ref[...], preferred_element_type=jnp.float32,
                precision=lax.Precision.HIGHEST)
        * dis_ref[:, 0:1]
    )


def _tc_mid(p0, p1, hp, dis8, b, w_next):
    return pl.pallas_call(
        _mid_body,
        grid=(GRID,),
        in_specs=[
            pl.BlockSpec((NC, ROW_BLK, DH), lambda r: (0, r, 0)),
            pl.BlockSpec((NC, ROW_BLK, DH), lambda r: (0, r, 0)),
            pl.BlockSpec((ROW_BLK, D), lambda r: (r, 0)),
            pl.BlockSpec((ROW_BLK, 8), lambda r: (r, 0)),
            pl.BlockSpec((1, D), lambda r: (0, 0)),
            pl.BlockSpec((D, D), lambda r: (0, 0)),
        ],
        out_specs=[
            pl.BlockSpec((ROW_BLK, D), lambda r: (r, 0)),
            pl.BlockSpec((ROW_BLK, D), lambda r: (r, 0)),
        ],
        out_shape=[
            jax.ShapeDtypeStruct((N, D), jnp.float32),
            jax.ShapeDtypeStruct((N, D), jnp.float32),
        ],
    )(p0, p1, hp, dis8, b, w_next)


def _last_body(p0_ref, p1_ref, hp_ref, dis_ref, b_ref, out_ref):
    t = _combine(p0_ref, p1_ref, hp_ref, dis_ref, b_ref)
    out_ref[...] = _log_softmax(t)


def _tc_last(p0, p1, hp, dis8, b):
    return pl.pallas_call(
        _last_body,
        grid=(GRID,),
        in_specs=[
            pl.BlockSpec((NC, ROW_BLK, DH), lambda r: (0, r, 0)),
            pl.BlockSpec((NC, ROW_BLK, DH), lambda r: (0, r, 0)),
            pl.BlockSpec((ROW_BLK, D), lambda r: (r, 0)),
            pl.BlockSpec((ROW_BLK, 8), lambda r: (r, 0)),
            pl.BlockSpec((1, D), lambda r: (0, 0)),
        ],
        out_specs=pl.BlockSpec((ROW_BLK, D), lambda r: (r, 0)),
        out_shape=jax.ShapeDtypeStruct((N, D), jnp.float32),
    )(p0, p1, hp, dis8, b)


def kernel(x, edge_index, W0, W1, W2, W3, W4, b0, b1, b2, b3, b4):
    Ws = [W0, W1, W2, W3, W4]
    bs = [b0, b1, b2, b3, b4]

    src = edge_index[0]
    dst = edge_index[1]
    pad = jnp.full((EPAD - E,), N, dtype=jnp.int32)
    src2 = jnp.concatenate([src, pad]).reshape(EB, CHUNK)
    dst2 = jnp.concatenate([dst, pad]).reshape(EB, CHUNK)

    ones_blk = jnp.ones((CHUNK, DW), dtype=jnp.float32)
    zeros_blk = jnp.zeros((CHUNK, DW), dtype=jnp.float32)
    zeros_blk_h = jnp.zeros((CHUNK, DH), dtype=jnp.float32)
    deg2 = _degree_call(dst2, ones_blk, zeros_blk)
    dis8, hp = _tc_first(deg2, x, Ws[0])

    zrows = jnp.zeros((NT - N, D), dtype=jnp.float32)
    outs = []
    for i in range(NUM_LAYERS):
        hp_pad = jnp.concatenate([hp, zrows])
        p0, p1 = _agg_call(hp_pad[:, :DH], hp_pad[:, DH:], src2, dst2,
                           zeros_blk_h)
        b = bs[i].reshape(1, D)
        if i < NUM_LAYERS - 1:
            o, hp = _tc_mid(p0, p1, hp, dis8, b, Ws[i + 1])
        else:
            o = _tc_last(p0, p1, hp, dis8, b)
        outs.append(o)
    return tuple(outs)
